# baseline (speedup 1.0000x reference)
import functools

import jax
import jax.numpy as jnp
from jax import lax
from jax.experimental import pallas as pl
from jax.experimental.pallas import tpu as pltpu
from jax.experimental.pallas import tpu_sc as plsc

N = 10000
E = 320000
H = 128
NP = 10240
EP = 327680
ER = EP // 128
RT2 = ER // 32
NPT = NP // 16

_MESH = plsc.VectorSubcoreMesh(core_axis_name="c", subcore_axis_name="s")


def _zero_rows128(ref, rows):
    zv = jnp.zeros((16,), jnp.float32)

    def body(i, _):
        for f in range(8):
            ref[i, pl.ds(f * 16, 16)] = zv
        return 0

    lax.fori_loop(0, rows, body, 0)


@functools.partial(
    pl.kernel,
    out_type=jax.ShapeDtypeStruct((2, NP, H), jnp.float32),
    mesh=_MESH,
    compiler_params=pltpu.CompilerParams(needs_layout_passes=False),
    scratch_types=[
        pltpu.VMEM((NP,), jnp.float32),
        pltpu.VMEM((8, 128), jnp.int32),
        pltpu.VMEM((8, 128), jnp.int32),
        pltpu.VMEM((8, 128), jnp.float32),
        pltpu.VMEM((128, H), jnp.float32),
        pltpu.VMEM((2, 16), jnp.float32),
        pltpu.VMEM_SHARED((NP, H), jnp.float32),
    ],
)
def _sc_a1(x_hbm, srcm, dstm, tm, scal_hbm, agg1_out,
           xv, eb_src, eb_dst, eb_t, valw, scalv, acc1w):
    cid = lax.axis_index("c")
    sid = lax.axis_index("s")
    wid = cid * 16 + sid

    pltpu.sync_copy(scal_hbm, scalv)
    pltpu.sync_copy(x_hbm, xv)

    _zero_rows128(valw, 128)

    def zacc(r, _):
        pltpu.sync_copy(valw, acc1w.at[pl.ds(sid * NPT + r * 128, 128), :])
        return 0

    lax.fori_loop(0, 5, zacc, 0)
    plsc.subcore_barrier()

    a1v = scalv[0, :]
    c1v = scalv[1, :]

    def p1b(jj, _):
        base = wid * RT2 + jj * 8
        pltpu.sync_copy(srcm.at[pl.ds(base, 8), :], eb_src)
        pltpu.sync_copy(dstm.at[pl.ds(base, 8), :], eb_dst)
        pltpu.sync_copy(tm.at[pl.ds(base, 8), :], eb_t)

        def p1(j, _):
            for g in range(8):
                srcv = eb_src[j, pl.ds(g * 16, 16)]
                tg = eb_t[j, pl.ds(g * 16, 16)]
                xg = plsc.load_gather(xv, [srcv])
                m = jnp.maximum(xg + a1v * tg + c1v, 0.0)
                for e in range(16):
                    valw[g * 16 + e, pl.ds(0, 16)] = jnp.full(
                        (16,), m[e], jnp.float32)
            pltpu.sync_copy(valw, acc1w.at[eb_dst.at[j]], add=True)
            return 0

        lax.fori_loop(0, 8, p1, 0)
        return 0

    lax.fori_loop(0, RT2 // 8, p1b, 0)
    plsc.subcore_barrier()

    def co(r, _):
        base = sid * NPT + r * 128
        pltpu.sync_copy(acc1w.at[pl.ds(base, 128), :],
                        agg1_out.at[cid, pl.ds(base, 128), :])
        return 0

    lax.fori_loop(0, 5, co, 0)


@functools.partial(
    pl.kernel,
    out_type=jax.ShapeDtypeStruct((2, NP, H), jnp.float32),
    mesh=_MESH,
    compiler_params=pltpu.CompilerParams(needs_layout_passes=False),
    scratch_types=[
        pltpu.VMEM((NP,), jnp.float32),
        pltpu.VMEM((4, 128), jnp.int32),
        pltpu.VMEM((4, 128), jnp.int32),
        pltpu.VMEM((4, 128), jnp.float32),
        pltpu.VMEM((128, H), jnp.float32),
        pltpu.VMEM((128, H), jnp.float32),
        pltpu.VMEM((3, H), jnp.float32),
        pltpu.VMEM_SHARED((NP, H), jnp.float32),
        pltpu.SemaphoreType.DMA,
        pltpu.SemaphoreType.DMA,
    ],
)
def _sc_a2(s_hbm, srcm, dstm, tm, cvec_hbm, agg2_out,
           sv, eb_src, eb_dst, eb_t, msg0, msg1, cvecv, acc2, sem0, sem1):
    cid = lax.axis_index("c")
    sid = lax.axis_index("s")
    wid = cid * 16 + sid

    pltpu.sync_copy(cvec_hbm, cvecv)
    pltpu.sync_copy(s_hbm, sv)
    _zero_rows128(msg0, 128)

    def zacc(r, _):
        pltpu.sync_copy(msg0, acc2.at[pl.ds(sid * NPT + r * 128, 128), :])
        return 0

    lax.fori_loop(0, 5, zacc, 0)
    plsc.subcore_barrier()

    w1v = [cvecv[0, pl.ds(f * 16, 16)] for f in range(8)]
    v2v = [cvecv[1, pl.ds(f * 16, 16)] for f in range(8)]
    u2v = [cvecv[2, pl.ds(f * 16, 16)] for f in range(8)]

    def compute(msg, j):
        def pg(g, _):
            srcv = eb_src[j, pl.ds(g * 16, 16)]
            svals = plsc.load_gather(sv, [srcv])
            tvals = eb_t[j, pl.ds(g * 16, 16)]
            for e in range(16):
                svb = jnp.full((16,), svals[e], jnp.float32)
                tvb = jnp.full((16,), tvals[e], jnp.float32)
                row = g * 16 + e
                for f in range(8):
                    m = jnp.maximum(
                        svb * w1v[f] + tvb * v2v[f] + u2v[f], 0.0)
                    msg[row, pl.ds(f * 16, 16)] = m
            return 0

        lax.fori_loop(0, 8, pg, 0)

    def p2b(jj, _):
        base = wid * RT2 + jj * 4
        pltpu.sync_copy(srcm.at[pl.ds(base, 4), :], eb_src)
        pltpu.sync_copy(dstm.at[pl.ds(base, 4), :], eb_dst)
        pltpu.sync_copy(tm.at[pl.ds(base, 4), :], eb_t)

        def pair(jp, _):
            j0 = 2 * jp
            j1 = 2 * jp + 1

            @pl.when(jp > 0)
            def _drain0():
                pltpu.make_async_copy(msg0, acc2.at[eb_dst.at[j0]],
                                      sem0).wait()

            compute(msg0, j0)
            pltpu.async_copy(msg0, acc2.at[eb_dst.at[j0]], sem0, add=True)

            @pl.when(jp > 0)
            def _drain1():
                pltpu.make_async_copy(msg1, acc2.at[eb_dst.at[j1]],
                                      sem1).wait()

            compute(msg1, j1)
            pltpu.async_copy(msg1, acc2.at[eb_dst.at[j1]], sem1, add=True)
            return 0

        lax.fori_loop(0, 2, pair, 0)
        pltpu.make_async_copy(msg0, acc2.at[eb_dst.at[2]], sem0).wait()
        pltpu.make_async_copy(msg1, acc2.at[eb_dst.at[3]], sem1).wait()
        return 0

    lax.fori_loop(0, RT2 // 4, p2b, 0)
    plsc.subcore_barrier()

    def co(r, _):
        base = sid * NPT + r * 128
        pltpu.sync_copy(acc2.at[pl.ds(base, 128), :],
                        agg2_out.at[cid, pl.ds(base, 128), :])
        return 0

    lax.fori_loop(0, 5, co, 0)


@functools.partial(
    pl.kernel,
    out_type=jax.ShapeDtypeStruct((2, NP, H), jnp.float32),
    mesh=_MESH,
    compiler_params=pltpu.CompilerParams(needs_layout_passes=False),
    scratch_types=[
        pltpu.VMEM((8, 128), jnp.int32),
        pltpu.VMEM((8, 128), jnp.int32),
        pltpu.VMEM((8, 128), jnp.float32),
        pltpu.VMEM((128, H), jnp.float32),
        pltpu.VMEM((128, H), jnp.float32),
        pltpu.VMEM((2, H), jnp.float32),
        pltpu.VMEM_SHARED((NP, H), jnp.float32),
        pltpu.SemaphoreType.DMA,
        pltpu.SemaphoreType.DMA,
    ],
)
def _sc_c(h2_hbm, srcm, dstm, tm, cvec_hbm, agg3_out,
          eb_src, eb_dst, eb_t, rows0, rows1, cvecv, acc3, sem0, sem1):
    cid = lax.axis_index("c")
    sid = lax.axis_index("s")
    wid = cid * 16 + sid

    pltpu.sync_copy(cvec_hbm, cvecv)
    _zero_rows128(rows0, 128)

    def zacc(r, _):
        pltpu.sync_copy(rows0, acc3.at[pl.ds(sid * NPT + r * 128, 128), :])
        return 0

    lax.fori_loop(0, 5, zacc, 0)
    plsc.subcore_barrier()

    v3v = [cvecv[0, pl.ds(f * 16, 16)] for f in range(8)]
    u3v = [cvecv[1, pl.ds(f * 16, 16)] for f in range(8)]

    def compute(rows, j):
        def pg(g, _):
            tvals = eb_t[j, pl.ds(g * 16, 16)]
            for e in range(16):
                tvb = jnp.full((16,), tvals[e], jnp.float32)
                row = g * 16 + e
                for f in range(8):
                    r = rows[row, pl.ds(f * 16, 16)]
                    rows[row, pl.ds(f * 16, 16)] = jnp.maximum(
                        r + tvb * v3v[f] + u3v[f], 0.0)
            return 0

        lax.fori_loop(0, 8, pg, 0)

    def p3b(jj, _):
        base = wid * RT2 + jj * 8
        pltpu.sync_copy(srcm.at[pl.ds(base, 8), :], eb_src)
        pltpu.sync_copy(dstm.at[pl.ds(base, 8), :], eb_dst)
        pltpu.sync_copy(tm.at[pl.ds(base, 8), :], eb_t)
        pltpu.async_copy(h2_hbm.at[eb_src.at[0]], rows0, sem0)

        def pair(jp, _):
            j0 = 2 * jp
            j1 = 2 * jp + 1
            pltpu.make_async_copy(
                h2_hbm.at[eb_src.at[j0]], rows0, sem0).wait()
            pltpu.async_copy(h2_hbm.at[eb_src.at[j1]], rows1, sem1)
            compute(rows0, j0)
            pltpu.sync_copy(rows0, acc3.at[eb_dst.at[j0]], add=True)
            pltpu.make_async_copy(
                h2_hbm.at[eb_src.at[j1]], rows1, sem1).wait()

            @pl.when(jp < 3)
            def _prefetch_even():
                pltpu.async_copy(h2_hbm.at[eb_src.at[j0 + 2]], rows0, sem0)

            compute(rows1, j1)
            pltpu.sync_copy(rows1, acc3.at[eb_dst.at[j1]], add=True)
            return 0

        lax.fori_loop(0, 4, pair, 0)
        return 0

    lax.fori_loop(0, RT2 // 8, p3b, 0)
    plsc.subcore_barrier()

    def co(r, _):
        base = sid * NPT + r * 128
        pltpu.sync_copy(acc3.at[pl.ds(base, 128), :],
                        agg3_out.at[cid, pl.ds(base, 128), :])
        return 0

    lax.fori_loop(0, 5, co, 0)


@functools.partial(
    pl.kernel,
    out_type=jax.ShapeDtypeStruct((ER, 128), jnp.float32),
    mesh=_MESH,
    compiler_params=pltpu.CompilerParams(needs_layout_passes=False),
    scratch_types=[
        pltpu.VMEM((NP,), jnp.float32),
        pltpu.VMEM((NP,), jnp.float32),
        pltpu.VMEM((8, 128), jnp.int32),
        pltpu.VMEM((8, 128), jnp.int32),
        pltpu.VMEM((8, 128), jnp.float32),
    ],
)
def _sc_e(p_hbm, q_hbm, srcm, dstm, out_hbm, pv, qv, eb_src, eb_dst, ob):
    cid = lax.axis_index("c")
    sid = lax.axis_index("s")
    wid = cid * 16 + sid

    pltpu.sync_copy(p_hbm, pv)
    pltpu.sync_copy(q_hbm, qv)

    def p4b(jj, _):
        base = wid * RT2 + jj * 8
        pltpu.sync_copy(srcm.at[pl.ds(base, 8), :], eb_src)
        pltpu.sync_copy(dstm.at[pl.ds(base, 8), :], eb_dst)

        def p4(j, _):
            for g in range(8):
                srcv = eb_src[j, pl.ds(g * 16, 16)]
                dstv = eb_dst[j, pl.ds(g * 16, 16)]
                ov = (plsc.load_gather(pv, [srcv])
                      + plsc.load_gather(qv, [dstv]))
                ob[j, pl.ds(g * 16, 16)] = ov
            return 0

        lax.fori_loop(0, 8, p4, 0)
        pltpu.sync_copy(ob, out_hbm.at[pl.ds(base, 8), :])
        return 0

    lax.fori_loop(0, RT2 // 8, p4b, 0)


_BLK = 1024


def _tc_s_body(x_ref, a0_ref, a1_ref, o_ref):
    o_ref[...] = x_ref[...] + a0_ref[...] + a1_ref[...]


def _tc_s(x2d, a0, a1):
    return pl.pallas_call(
        _tc_s_body,
        out_shape=jax.ShapeDtypeStruct((NP // 128, 128), jnp.float32),
    )(x2d, a0, a1)


def _tc_b_body(s_ref, w1_ref, bn1_ref, a0_ref, a1_ref, w2_ref, bn2_ref, o_ref):
    z = s_ref[...] * w1_ref[...] + bn1_ref[...] + a0_ref[0] + a1_ref[0]
    o_ref[...] = jax.lax.dot_general(
        z, w2_ref[...], (((1,), (0,)), ((), ())),
        preferred_element_type=jnp.float32,
        precision=jax.lax.Precision.HIGHEST) + bn2_ref[...]


def _tc_b(s2d, w1r, bn1r, agg2, W2, bn2r):
    grid = (NP // _BLK,)
    return pl.pallas_call(
        _tc_b_body,
        grid=grid,
        in_specs=[
            pl.BlockSpec((_BLK, 1), lambda i: (i, 0)),
            pl.BlockSpec((1, H), lambda i: (0, 0)),
            pl.BlockSpec((1, H), lambda i: (0, 0)),
            pl.BlockSpec((1, _BLK, H), lambda i: (0, i, 0)),
            pl.BlockSpec((1, _BLK, H), lambda i: (1, i, 0)),
            pl.BlockSpec((H, H), lambda i: (0, 0)),
            pl.BlockSpec((1, H), lambda i: (0, 0)),
        ],
        out_specs=pl.BlockSpec((_BLK, H), lambda i: (i, 0)),
        out_shape=jax.ShapeDtypeStruct((NP, H), jnp.float32),
    )(s2d, w1r, bn1r, agg2, agg2, W2, bn2r)


def _tc_d_body(h2_ref, a0_ref, a1_ref, g1_ref, g2_ref, dc_ref, p_ref, q_ref):
    y = h2_ref[...] + a0_ref[0] + a1_ref[0]
    p_ref[...] = jnp.sum(y * g1_ref[...], axis=1, keepdims=True)
    q_ref[...] = jnp.sum(y * g2_ref[...], axis=1, keepdims=True) + dc_ref[0, 0]


def _tc_d(h2, agg3, g1r, g2r, dc):
    grid = (NP // _BLK,)
    return pl.pallas_call(
        _tc_d_body,
        grid=grid,
        in_specs=[
            pl.BlockSpec((_BLK, H), lambda i: (i, 0)),
            pl.BlockSpec((1, _BLK, H), lambda i: (0, i, 0)),
            pl.BlockSpec((1, _BLK, H), lambda i: (1, i, 0)),
            pl.BlockSpec((1, H), lambda i: (0, 0)),
            pl.BlockSpec((1, H), lambda i: (0, 0)),
            pl.BlockSpec((1, 1), lambda i: (0, 0)),
        ],
        out_specs=[
            pl.BlockSpec((_BLK, 1), lambda i: (i, 0)),
            pl.BlockSpec((_BLK, 1), lambda i: (i, 0)),
        ],
        out_shape=[
            jax.ShapeDtypeStruct((NP, 1), jnp.float32),
            jax.ShapeDtypeStruct((NP, 1), jnp.float32),
        ],
    )(h2, agg3, agg3, g1r, g2r, dc)


def kernel(x, edge_index, edge_attr, W_em, b_em, W_le1, b_le1, W_nn1, b_nn1,
           W_le2, b_le2, W_nn2, b_nn2, W_le3, b_le3, W_nn3, b_nn3,
           W_dec, b_dec):
    src = edge_index[0].astype(jnp.int32)
    dst = edge_index[1].astype(jnp.int32)
    t = edge_attr[:, 0]

    xp = jnp.pad(x[:, 0], (0, NP - N))
    pad = EP - E
    srcm = jnp.concatenate([src, jnp.zeros((pad,), jnp.int32)]).reshape(ER, 128)
    dpad = N + (jnp.arange(pad, dtype=jnp.int32) % 32)
    dstm = jnp.concatenate([dst, dpad]).reshape(ER, 128)
    tm = jnp.concatenate([t, jnp.zeros((pad,), jnp.float32)]).reshape(ER, 128)

    a1 = (W_em @ W_le1)[0, 0]
    c1 = (b_em @ W_le1 + b_le1)[0]
    scal = jnp.stack([jnp.full((16,), a1, jnp.float32),
                      jnp.full((16,), c1, jnp.float32)])
    w1 = W_nn1[0]
    v2 = (W_em @ W_le2)[0]
    u2 = b_nn1 + b_em @ W_le2 + b_le2
    cvec2 = jnp.stack([w1, v2, u2])
    v3 = (W_em @ W_le3)[0]
    u3 = b_em @ W_le3 + b_le3
    cvec3 = jnp.stack([v3, u3])
    g1 = W_nn3 @ W_dec[:H, 0]
    g2 = W_nn3 @ W_dec[H:, 0]
    dconst = b_nn3 @ (W_dec[:H, 0] + W_dec[H:, 0]) + b_dec[0]

    agg1 = _sc_a1(xp, srcm, dstm, tm, scal)
    s2d = _tc_s(xp.reshape(NP // 128, 128),
                agg1[0, :, 0].reshape(NP // 128, 128),
                agg1[1, :, 0].reshape(NP // 128, 128))
    s_vec = s2d.reshape(NP)
    agg2 = _sc_a2(s_vec, srcm, dstm, tm, cvec2)
    h2 = _tc_b(s_vec[:, None], w1[None, :], b_nn1[None, :], agg2,
               W_nn2, b_nn2[None, :])
    agg3 = _sc_c(h2, srcm, dstm, tm, cvec3)
    pcol, qcol = _tc_d(h2, agg3, g1[None, :], g2[None, :],
                       jnp.full((1, 1), dconst, jnp.float32))
    out2d = _sc_e(pcol[:, 0], qcol[:, 0], srcm, dstm)
    return out2d.reshape(EP)[:E][:, None]


# speedup vs baseline: 1.0113x; 1.0113x over previous
import functools

import jax
import jax.numpy as jnp
from jax import lax
from jax.experimental import pallas as pl
from jax.experimental.pallas import tpu as pltpu
from jax.experimental.pallas import tpu_sc as plsc

N = 10000
E = 320000
H = 128
NP = 10240
EP = 327680
ER = EP // 128
RT2 = ER // 32
NPT = NP // 16

_MESH = plsc.VectorSubcoreMesh(core_axis_name="c", subcore_axis_name="s")


def _zero_rows128(ref, rows):
    zv = jnp.zeros((16,), jnp.float32)

    def body(i, _):
        for f in range(8):
            ref[i, pl.ds(f * 16, 16)] = zv
        return 0

    lax.fori_loop(0, rows, body, 0)


@functools.partial(
    pl.kernel,
    out_type=jax.ShapeDtypeStruct((2, NP, H), jnp.float32),
    mesh=_MESH,
    compiler_params=pltpu.CompilerParams(needs_layout_passes=False),
    scratch_types=[
        pltpu.VMEM((NP,), jnp.float32),
        pltpu.VMEM((8, 128), jnp.int32),
        pltpu.VMEM((8, 128), jnp.int32),
        pltpu.VMEM((8, 128), jnp.float32),
        pltpu.VMEM((128, H), jnp.float32),
        pltpu.VMEM((2, 16), jnp.float32),
        pltpu.VMEM_SHARED((NP, H), jnp.float32),
    ],
)
def _sc_a1(x_hbm, srcm, dstm, tm, scal_hbm, agg1_out,
           xv, eb_src, eb_dst, eb_t, valw, scalv, acc1w):
    cid = lax.axis_index("c")
    sid = lax.axis_index("s")
    wid = cid * 16 + sid

    pltpu.sync_copy(scal_hbm, scalv)
    pltpu.sync_copy(x_hbm, xv)

    _zero_rows128(valw, 128)

    def zacc(r, _):
        pltpu.sync_copy(valw, acc1w.at[pl.ds(sid * NPT + r * 128, 128), :])
        return 0

    lax.fori_loop(0, 5, zacc, 0)
    plsc.subcore_barrier()

    a1v = scalv[0, :]
    c1v = scalv[1, :]

    def p1b(jj, _):
        base = wid * RT2 + jj * 8
        pltpu.sync_copy(srcm.at[pl.ds(base, 8), :], eb_src)
        pltpu.sync_copy(dstm.at[pl.ds(base, 8), :], eb_dst)
        pltpu.sync_copy(tm.at[pl.ds(base, 8), :], eb_t)

        def p1(j, _):
            for g in range(8):
                srcv = eb_src[j, pl.ds(g * 16, 16)]
                tg = eb_t[j, pl.ds(g * 16, 16)]
                xg = plsc.load_gather(xv, [srcv])
                m = jnp.maximum(xg + a1v * tg + c1v, 0.0)
                for e in range(16):
                    valw[g * 16 + e, pl.ds(0, 16)] = jnp.full(
                        (16,), m[e], jnp.float32)
            pltpu.sync_copy(valw, acc1w.at[eb_dst.at[j]], add=True)
            return 0

        lax.fori_loop(0, 8, p1, 0)
        return 0

    lax.fori_loop(0, RT2 // 8, p1b, 0)
    plsc.subcore_barrier()

    def co(r, _):
        base = sid * NPT + r * 128
        pltpu.sync_copy(acc1w.at[pl.ds(base, 128), :],
                        agg1_out.at[cid, pl.ds(base, 128), :])
        return 0

    lax.fori_loop(0, 5, co, 0)


@functools.partial(
    pl.kernel,
    out_type=jax.ShapeDtypeStruct((2, NP, H), jnp.float32),
    mesh=_MESH,
    compiler_params=pltpu.CompilerParams(needs_layout_passes=False),
    scratch_types=[
        pltpu.VMEM((NP,), jnp.float32),
        pltpu.VMEM((8, 128), jnp.int32),
        pltpu.VMEM((8, 128), jnp.int32),
        pltpu.VMEM((8, 128), jnp.float32),
        pltpu.VMEM((128, H), jnp.float32),
        pltpu.VMEM((3, H), jnp.float32),
        pltpu.VMEM_SHARED((NP, H), jnp.float32),
    ],
)
def _sc_a2(s_hbm, srcm, dstm, tm, cvec_hbm, agg2_out,
           sv, eb_src, eb_dst, eb_t, msg0, cvecv, acc2):
    cid = lax.axis_index("c")
    sid = lax.axis_index("s")
    wid = cid * 16 + sid

    pltpu.sync_copy(cvec_hbm, cvecv)
    pltpu.sync_copy(s_hbm, sv)
    _zero_rows128(msg0, 128)

    def zacc(r, _):
        pltpu.sync_copy(msg0, acc2.at[pl.ds(sid * NPT + r * 128, 128), :])
        return 0

    lax.fori_loop(0, 5, zacc, 0)
    plsc.subcore_barrier()

    w1v = [cvecv[0, pl.ds(f * 16, 16)] for f in range(8)]
    v2v = [cvecv[1, pl.ds(f * 16, 16)] for f in range(8)]
    u2v = [cvecv[2, pl.ds(f * 16, 16)] for f in range(8)]

    def compute(msg, j):
        def pg(g, _):
            srcv = eb_src[j, pl.ds(g * 16, 16)]
            svals = plsc.load_gather(sv, [srcv])
            tvals = eb_t[j, pl.ds(g * 16, 16)]
            for e in range(16):
                svb = jnp.full((16,), svals[e], jnp.float32)
                tvb = jnp.full((16,), tvals[e], jnp.float32)
                row = g * 16 + e
                for f in range(8):
                    m = jnp.maximum(
                        svb * w1v[f] + tvb * v2v[f] + u2v[f], 0.0)
                    msg[row, pl.ds(f * 16, 16)] = m
            return 0

        lax.fori_loop(0, 8, pg, 0)

    def p2b(jj, _):
        base = wid * RT2 + jj * 8
        pltpu.sync_copy(srcm.at[pl.ds(base, 8), :], eb_src)
        pltpu.sync_copy(dstm.at[pl.ds(base, 8), :], eb_dst)
        pltpu.sync_copy(tm.at[pl.ds(base, 8), :], eb_t)

        def p2(j, _):
            compute(msg0, j)
            pltpu.sync_copy(msg0, acc2.at[eb_dst.at[j]], add=True)
            return 0

        lax.fori_loop(0, 8, p2, 0)
        return 0

    lax.fori_loop(0, RT2 // 8, p2b, 0)
    plsc.subcore_barrier()

    def co(r, _):
        base = sid * NPT + r * 128
        pltpu.sync_copy(acc2.at[pl.ds(base, 128), :],
                        agg2_out.at[cid, pl.ds(base, 128), :])
        return 0

    lax.fori_loop(0, 5, co, 0)


@functools.partial(
    pl.kernel,
    out_type=jax.ShapeDtypeStruct((2, NP, H), jnp.float32),
    mesh=_MESH,
    compiler_params=pltpu.CompilerParams(needs_layout_passes=False),
    scratch_types=[
        pltpu.VMEM((8, 128), jnp.int32),
        pltpu.VMEM((8, 128), jnp.int32),
        pltpu.VMEM((8, 128), jnp.float32),
        pltpu.VMEM((128, H), jnp.float32),
        pltpu.VMEM((128, H), jnp.float32),
        pltpu.VMEM((2, H), jnp.float32),
        pltpu.VMEM_SHARED((NP, H), jnp.float32),
        pltpu.SemaphoreType.DMA,
        pltpu.SemaphoreType.DMA,
    ],
)
def _sc_c(h2_hbm, srcm, dstm, tm, cvec_hbm, agg3_out,
          eb_src, eb_dst, eb_t, rows0, rows1, cvecv, acc3, sem0, sem1):
    cid = lax.axis_index("c")
    sid = lax.axis_index("s")
    wid = cid * 16 + sid

    pltpu.sync_copy(cvec_hbm, cvecv)
    _zero_rows128(rows0, 128)

    def zacc(r, _):
        pltpu.sync_copy(rows0, acc3.at[pl.ds(sid * NPT + r * 128, 128), :])
        return 0

    lax.fori_loop(0, 5, zacc, 0)
    plsc.subcore_barrier()

    v3v = [cvecv[0, pl.ds(f * 16, 16)] for f in range(8)]
    u3v = [cvecv[1, pl.ds(f * 16, 16)] for f in range(8)]

    def compute(rows, j):
        def pg(g, _):
            tvals = eb_t[j, pl.ds(g * 16, 16)]
            for e in range(16):
                tvb = jnp.full((16,), tvals[e], jnp.float32)
                row = g * 16 + e
                for f in range(8):
                    r = rows[row, pl.ds(f * 16, 16)]
                    rows[row, pl.ds(f * 16, 16)] = jnp.maximum(
                        r + tvb * v3v[f] + u3v[f], 0.0)
            return 0

        lax.fori_loop(0, 8, pg, 0)

    def p3b(jj, _):
        base = wid * RT2 + jj * 8
        pltpu.sync_copy(srcm.at[pl.ds(base, 8), :], eb_src)
        pltpu.sync_copy(dstm.at[pl.ds(base, 8), :], eb_dst)
        pltpu.sync_copy(tm.at[pl.ds(base, 8), :], eb_t)
        pltpu.async_copy(h2_hbm.at[eb_src.at[0]], rows0, sem0)

        def pair(jp, _):
            j0 = 2 * jp
            j1 = 2 * jp + 1
            pltpu.make_async_copy(
                h2_hbm.at[eb_src.at[j0]], rows0, sem0).wait()
            pltpu.async_copy(h2_hbm.at[eb_src.at[j1]], rows1, sem1)
            compute(rows0, j0)
            pltpu.sync_copy(rows0, acc3.at[eb_dst.at[j0]], add=True)
            pltpu.make_async_copy(
                h2_hbm.at[eb_src.at[j1]], rows1, sem1).wait()

            @pl.when(jp < 3)
            def _prefetch_even():
                pltpu.async_copy(h2_hbm.at[eb_src.at[j0 + 2]], rows0, sem0)

            compute(rows1, j1)
            pltpu.sync_copy(rows1, acc3.at[eb_dst.at[j1]], add=True)
            return 0

        lax.fori_loop(0, 4, pair, 0)
        return 0

    lax.fori_loop(0, RT2 // 8, p3b, 0)
    plsc.subcore_barrier()

    def co(r, _):
        base = sid * NPT + r * 128
        pltpu.sync_copy(acc3.at[pl.ds(base, 128), :],
                        agg3_out.at[cid, pl.ds(base, 128), :])
        return 0

    lax.fori_loop(0, 5, co, 0)


@functools.partial(
    pl.kernel,
    out_type=jax.ShapeDtypeStruct((ER, 128), jnp.float32),
    mesh=_MESH,
    compiler_params=pltpu.CompilerParams(needs_layout_passes=False),
    scratch_types=[
        pltpu.VMEM((NP,), jnp.float32),
        pltpu.VMEM((NP,), jnp.float32),
        pltpu.VMEM((8, 128), jnp.int32),
        pltpu.VMEM((8, 128), jnp.int32),
        pltpu.VMEM((8, 128), jnp.float32),
    ],
)
def _sc_e(p_hbm, q_hbm, srcm, dstm, out_hbm, pv, qv, eb_src, eb_dst, ob):
    cid = lax.axis_index("c")
    sid = lax.axis_index("s")
    wid = cid * 16 + sid

    pltpu.sync_copy(p_hbm, pv)
    pltpu.sync_copy(q_hbm, qv)

    def p4b(jj, _):
        base = wid * RT2 + jj * 8
        pltpu.sync_copy(srcm.at[pl.ds(base, 8), :], eb_src)
        pltpu.sync_copy(dstm.at[pl.ds(base, 8), :], eb_dst)

        def p4(j, _):
            for g in range(8):
                srcv = eb_src[j, pl.ds(g * 16, 16)]
                dstv = eb_dst[j, pl.ds(g * 16, 16)]
                ov = (plsc.load_gather(pv, [srcv])
                      + plsc.load_gather(qv, [dstv]))
                ob[j, pl.ds(g * 16, 16)] = ov
            return 0

        lax.fori_loop(0, 8, p4, 0)
        pltpu.sync_copy(ob, out_hbm.at[pl.ds(base, 8), :])
        return 0

    lax.fori_loop(0, RT2 // 8, p4b, 0)


_BLK = 1024


def _tc_s_body(x_ref, a0_ref, a1_ref, o_ref):
    o_ref[...] = x_ref[...] + a0_ref[...] + a1_ref[...]


def _tc_s(x2d, a0, a1):
    return pl.pallas_call(
        _tc_s_body,
        out_shape=jax.ShapeDtypeStruct((NP // 128, 128), jnp.float32),
    )(x2d, a0, a1)


def _tc_b_body(s_ref, w1_ref, bn1_ref, a0_ref, a1_ref, w2_ref, bn2_ref, o_ref):
    z = s_ref[...] * w1_ref[...] + bn1_ref[...] + a0_ref[0] + a1_ref[0]
    o_ref[...] = jax.lax.dot_general(
        z, w2_ref[...], (((1,), (0,)), ((), ())),
        preferred_element_type=jnp.float32,
        precision=jax.lax.Precision.HIGHEST) + bn2_ref[...]


def _tc_b(s2d, w1r, bn1r, agg2, W2, bn2r):
    grid = (NP // _BLK,)
    return pl.pallas_call(
        _tc_b_body,
        grid=grid,
        in_specs=[
            pl.BlockSpec((_BLK, 1), lambda i: (i, 0)),
            pl.BlockSpec((1, H), lambda i: (0, 0)),
            pl.BlockSpec((1, H), lambda i: (0, 0)),
            pl.BlockSpec((1, _BLK, H), lambda i: (0, i, 0)),
            pl.BlockSpec((1, _BLK, H), lambda i: (1, i, 0)),
            pl.BlockSpec((H, H), lambda i: (0, 0)),
            pl.BlockSpec((1, H), lambda i: (0, 0)),
        ],
        out_specs=pl.BlockSpec((_BLK, H), lambda i: (i, 0)),
        out_shape=jax.ShapeDtypeStruct((NP, H), jnp.float32),
    )(s2d, w1r, bn1r, agg2, agg2, W2, bn2r)


def _tc_d_body(h2_ref, a0_ref, a1_ref, g1_ref, g2_ref, dc_ref, p_ref, q_ref):
    y = h2_ref[...] + a0_ref[0] + a1_ref[0]
    p_ref[...] = jnp.sum(y * g1_ref[...], axis=1, keepdims=True)
    q_ref[...] = jnp.sum(y * g2_ref[...], axis=1, keepdims=True) + dc_ref[0, 0]


def _tc_d(h2, agg3, g1r, g2r, dc):
    grid = (NP // _BLK,)
    return pl.pallas_call(
        _tc_d_body,
        grid=grid,
        in_specs=[
            pl.BlockSpec((_BLK, H), lambda i: (i, 0)),
            pl.BlockSpec((1, _BLK, H), lambda i: (0, i, 0)),
            pl.BlockSpec((1, _BLK, H), lambda i: (1, i, 0)),
            pl.BlockSpec((1, H), lambda i: (0, 0)),
            pl.BlockSpec((1, H), lambda i: (0, 0)),
            pl.BlockSpec((1, 1), lambda i: (0, 0)),
        ],
        out_specs=[
            pl.BlockSpec((_BLK, 1), lambda i: (i, 0)),
            pl.BlockSpec((_BLK, 1), lambda i: (i, 0)),
        ],
        out_shape=[
            jax.ShapeDtypeStruct((NP, 1), jnp.float32),
            jax.ShapeDtypeStruct((NP, 1), jnp.float32),
        ],
    )(h2, agg3, agg3, g1r, g2r, dc)


def kernel(x, edge_index, edge_attr, W_em, b_em, W_le1, b_le1, W_nn1, b_nn1,
           W_le2, b_le2, W_nn2, b_nn2, W_le3, b_le3, W_nn3, b_nn3,
           W_dec, b_dec):
    src = edge_index[0].astype(jnp.int32)
    dst = edge_index[1].astype(jnp.int32)
    t = edge_attr[:, 0]

    xp = jnp.pad(x[:, 0], (0, NP - N))
    pad = EP - E
    srcm = jnp.concatenate([src, jnp.zeros((pad,), jnp.int32)]).reshape(ER, 128)
    dpad = N + (jnp.arange(pad, dtype=jnp.int32) % 32)
    dstm = jnp.concatenate([dst, dpad]).reshape(ER, 128)
    tm = jnp.concatenate([t, jnp.zeros((pad,), jnp.float32)]).reshape(ER, 128)

    a1 = (W_em @ W_le1)[0, 0]
    c1 = (b_em @ W_le1 + b_le1)[0]
    scal = jnp.stack([jnp.full((16,), a1, jnp.float32),
                      jnp.full((16,), c1, jnp.float32)])
    w1 = W_nn1[0]
    v2 = (W_em @ W_le2)[0]
    u2 = b_nn1 + b_em @ W_le2 + b_le2
    cvec2 = jnp.stack([w1, v2, u2])
    v3 = (W_em @ W_le3)[0]
    u3 = b_em @ W_le3 + b_le3
    cvec3 = jnp.stack([v3, u3])
    g1 = W_nn3 @ W_dec[:H, 0]
    g2 = W_nn3 @ W_dec[H:, 0]
    dconst = b_nn3 @ (W_dec[:H, 0] + W_dec[H:, 0]) + b_dec[0]

    agg1 = _sc_a1(xp, srcm, dstm, tm, scal)
    s2d = _tc_s(xp.reshape(NP // 128, 128),
                agg1[0, :, 0].reshape(NP // 128, 128),
                agg1[1, :, 0].reshape(NP // 128, 128))
    s_vec = s2d.reshape(NP)
    agg2 = _sc_a2(s_vec, srcm, dstm, tm, cvec2)
    h2 = _tc_b(s_vec[:, None], w1[None, :], b_nn1[None, :], agg2,
               W_nn2, b_nn2[None, :])
    agg3 = _sc_c(h2, srcm, dstm, tm, cvec3)
    pcol, qcol = _tc_d(h2, agg3, g1[None, :], g2[None, :],
                       jnp.full((1, 1), dconst, jnp.float32))
    out2d = _sc_e(pcol[:, 0], qcol[:, 0], srcm, dstm)
    return out2d.reshape(EP)[:E][:, None]


# speedup vs baseline: 1.0992x; 1.0869x over previous
import functools

import jax
import jax.numpy as jnp
from jax import lax
from jax.experimental import pallas as pl
from jax.experimental.pallas import tpu as pltpu
from jax.experimental.pallas import tpu_sc as plsc

N = 10000
E = 320000
H = 128
NP = 10240
EP = 327680
ER = EP // 128
RT2 = ER // 32
NPT = NP // 16

_MESH = plsc.VectorSubcoreMesh(core_axis_name="c", subcore_axis_name="s")


def _zero_rows128(ref, rows):
    zv = jnp.zeros((16,), jnp.float32)

    def body(i, _):
        for f in range(8):
            ref[i, pl.ds(f * 16, 16)] = zv
        return 0

    lax.fori_loop(0, rows, body, 0)


@functools.partial(
    pl.kernel,
    out_type=jax.ShapeDtypeStruct((2, NP, H), jnp.float32),
    mesh=_MESH,
    compiler_params=pltpu.CompilerParams(needs_layout_passes=False),
    scratch_types=[
        pltpu.VMEM((NP,), jnp.float32),
        pltpu.VMEM((8, 128), jnp.int32),
        pltpu.VMEM((8, 128), jnp.int32),
        pltpu.VMEM((8, 128), jnp.float32),
        pltpu.VMEM((128, H), jnp.float32),
        pltpu.VMEM((2, 16), jnp.float32),
        pltpu.VMEM_SHARED((NP, H), jnp.float32),
    ],
)
def _sc_a1(x_hbm, srcm, dstm, tm, scal_hbm, agg1_out,
           xv, eb_src, eb_dst, eb_t, valw, scalv, acc1w):
    cid = lax.axis_index("c")
    sid = lax.axis_index("s")
    wid = cid * 16 + sid

    pltpu.sync_copy(scal_hbm, scalv)
    pltpu.sync_copy(x_hbm, xv)

    _zero_rows128(valw, 128)

    def zacc(r, _):
        pltpu.sync_copy(valw, acc1w.at[pl.ds(sid * NPT + r * 128, 128), :])
        return 0

    lax.fori_loop(0, 5, zacc, 0)
    plsc.subcore_barrier()

    a1v = scalv[0, :]
    c1v = scalv[1, :]

    def p1b(jj, _):
        base = wid * RT2 + jj * 8
        pltpu.sync_copy(srcm.at[pl.ds(base, 8), :], eb_src)
        pltpu.sync_copy(dstm.at[pl.ds(base, 8), :], eb_dst)
        pltpu.sync_copy(tm.at[pl.ds(base, 8), :], eb_t)

        def p1(j, _):
            for g in range(8):
                srcv = eb_src[j, pl.ds(g * 16, 16)]
                tg = eb_t[j, pl.ds(g * 16, 16)]
                xg = plsc.load_gather(xv, [srcv])
                m = jnp.maximum(xg + a1v * tg + c1v, 0.0)
                for e in range(16):
                    valw[g * 16 + e, pl.ds(0, 16)] = jnp.full(
                        (16,), m[e], jnp.float32)
            pltpu.sync_copy(valw, acc1w.at[eb_dst.at[j]], add=True)
            return 0

        lax.fori_loop(0, 8, p1, 0)
        return 0

    lax.fori_loop(0, RT2 // 8, p1b, 0)
    plsc.subcore_barrier()

    def co(r, _):
        base = sid * NPT + r * 128
        pltpu.sync_copy(acc1w.at[pl.ds(base, 128), :],
                        agg1_out.at[cid, pl.ds(base, 128), :])
        return 0

    lax.fori_loop(0, 5, co, 0)


@functools.partial(
    pl.kernel,
    out_type=jax.ShapeDtypeStruct((2, NP, H), jnp.float32),
    mesh=_MESH,
    compiler_params=pltpu.CompilerParams(needs_layout_passes=False),
    scratch_types=[
        pltpu.VMEM((NP,), jnp.float32),
        pltpu.VMEM((8, 128), jnp.int32),
        pltpu.VMEM((8, 128), jnp.int32),
        pltpu.VMEM((8, 128), jnp.float32),
        pltpu.VMEM((128, H), jnp.float32),
        pltpu.VMEM((3, H), jnp.float32),
        pltpu.VMEM_SHARED((NP, H), jnp.float32),
    ],
)
def _sc_a2(s_hbm, srcm, dstm, tm, cvec_hbm, agg2_out,
           sv, eb_src, eb_dst, eb_t, msg0, cvecv, acc2):
    cid = lax.axis_index("c")
    sid = lax.axis_index("s")
    wid = cid * 16 + sid

    pltpu.sync_copy(cvec_hbm, cvecv)
    pltpu.sync_copy(s_hbm, sv)
    _zero_rows128(msg0, 128)

    def zacc(r, _):
        pltpu.sync_copy(msg0, acc2.at[pl.ds(sid * NPT + r * 128, 128), :])
        return 0

    lax.fori_loop(0, 5, zacc, 0)
    plsc.subcore_barrier()

    w1v = [cvecv[0, pl.ds(f * 16, 16)] for f in range(8)]
    v2v = [cvecv[1, pl.ds(f * 16, 16)] for f in range(8)]
    u2v = [cvecv[2, pl.ds(f * 16, 16)] for f in range(8)]

    def compute(msg, j):
        def pg(g, _):
            srcv = eb_src[j, pl.ds(g * 16, 16)]
            svals = plsc.load_gather(sv, [srcv])
            tvals = eb_t[j, pl.ds(g * 16, 16)]
            for e in range(16):
                svb = jnp.full((16,), svals[e], jnp.float32)
                tvb = jnp.full((16,), tvals[e], jnp.float32)
                row = g * 16 + e
                for f in range(8):
                    m = jnp.maximum(
                        svb * w1v[f] + tvb * v2v[f] + u2v[f], 0.0)
                    msg[row, pl.ds(f * 16, 16)] = m
            return 0

        lax.fori_loop(0, 8, pg, 0)

    def p2b(jj, _):
        base = wid * RT2 + jj * 8
        pltpu.sync_copy(srcm.at[pl.ds(base, 8), :], eb_src)
        pltpu.sync_copy(dstm.at[pl.ds(base, 8), :], eb_dst)
        pltpu.sync_copy(tm.at[pl.ds(base, 8), :], eb_t)

        def p2(j, _):
            compute(msg0, j)
            pltpu.sync_copy(msg0, acc2.at[eb_dst.at[j]], add=True)
            return 0

        lax.fori_loop(0, 8, p2, 0)
        return 0

    lax.fori_loop(0, RT2 // 8, p2b, 0)
    plsc.subcore_barrier()

    def co(r, _):
        base = sid * NPT + r * 128
        pltpu.sync_copy(acc2.at[pl.ds(base, 128), :],
                        agg2_out.at[cid, pl.ds(base, 128), :])
        return 0

    lax.fori_loop(0, 5, co, 0)


@functools.partial(
    pl.kernel,
    out_type=jax.ShapeDtypeStruct((2, NP, H), jnp.float32),
    mesh=_MESH,
    compiler_params=pltpu.CompilerParams(needs_layout_passes=False),
    scratch_types=[
        pltpu.VMEM((8, 128), jnp.int32),
        pltpu.VMEM((8, 128), jnp.int32),
        pltpu.VMEM((8, 128), jnp.float32),
        pltpu.VMEM((128, H), jnp.float32),
        pltpu.VMEM((128, H), jnp.float32),
        pltpu.VMEM((2, H), jnp.float32),
        pltpu.VMEM_SHARED((NP, H), jnp.float32),
        pltpu.SemaphoreType.DMA,
        pltpu.SemaphoreType.DMA,
    ],
)
def _sc_c(h2_hbm, srcm, dstm, tm, cvec_hbm, agg3_out,
          eb_src, eb_dst, eb_t, rows0, rows1, cvecv, acc3, sem0, sem1):
    cid = lax.axis_index("c")
    sid = lax.axis_index("s")
    wid = cid * 16 + sid

    pltpu.sync_copy(cvec_hbm, cvecv)
    _zero_rows128(rows0, 128)

    def zacc(r, _):
        pltpu.sync_copy(rows0, acc3.at[pl.ds(sid * NPT + r * 128, 128), :])
        return 0

    lax.fori_loop(0, 5, zacc, 0)
    plsc.subcore_barrier()

    v3v = [cvecv[0, pl.ds(f * 16, 16)] for f in range(8)]
    u3v = [cvecv[1, pl.ds(f * 16, 16)] for f in range(8)]

    nrows = jnp.where(cid == 0, 120, 40)
    tstart = cid * (16 * 120) + sid * nrows

    def compute(rows, j):
        def pg(g, _):
            tvals = eb_t[j, pl.ds(g * 16, 16)]
            for e in range(16):
                tvb = jnp.full((16,), tvals[e], jnp.float32)
                row = g * 16 + e
                for f in range(8):
                    r = rows[row, pl.ds(f * 16, 16)]
                    rows[row, pl.ds(f * 16, 16)] = jnp.maximum(
                        r + tvb * v3v[f] + u3v[f], 0.0)
            return 0

        lax.fori_loop(0, 8, pg, 0)

    def p3b(jj, _):
        base = tstart + jj * 8
        pltpu.sync_copy(srcm.at[pl.ds(base, 8), :], eb_src)
        pltpu.sync_copy(dstm.at[pl.ds(base, 8), :], eb_dst)
        pltpu.sync_copy(tm.at[pl.ds(base, 8), :], eb_t)
        pltpu.async_copy(h2_hbm.at[eb_src.at[0]], rows0, sem0)

        def pair(jp, _):
            j0 = 2 * jp
            j1 = 2 * jp + 1
            pltpu.make_async_copy(
                h2_hbm.at[eb_src.at[j0]], rows0, sem0).wait()
            pltpu.async_copy(h2_hbm.at[eb_src.at[j1]], rows1, sem1)
            compute(rows0, j0)
            pltpu.sync_copy(rows0, acc3.at[eb_dst.at[j0]], add=True)
            pltpu.make_async_copy(
                h2_hbm.at[eb_src.at[j1]], rows1, sem1).wait()

            @pl.when(jp < 3)
            def _prefetch_even():
                pltpu.async_copy(h2_hbm.at[eb_src.at[j0 + 2]], rows0, sem0)

            compute(rows1, j1)
            pltpu.sync_copy(rows1, acc3.at[eb_dst.at[j1]], add=True)
            return 0

        lax.fori_loop(0, 4, pair, 0)
        return 0

    lax.fori_loop(0, nrows // 8, p3b, 0)
    plsc.subcore_barrier()

    def co(r, _):
        base = sid * NPT + r * 128
        pltpu.sync_copy(acc3.at[pl.ds(base, 128), :],
                        agg3_out.at[cid, pl.ds(base, 128), :])
        return 0

    lax.fori_loop(0, 5, co, 0)


@functools.partial(
    pl.kernel,
    out_type=jax.ShapeDtypeStruct((ER, 128), jnp.float32),
    mesh=_MESH,
    compiler_params=pltpu.CompilerParams(needs_layout_passes=False),
    scratch_types=[
        pltpu.VMEM((NP,), jnp.float32),
        pltpu.VMEM((NP,), jnp.float32),
        pltpu.VMEM((8, 128), jnp.int32),
        pltpu.VMEM((8, 128), jnp.int32),
        pltpu.VMEM((8, 128), jnp.float32),
    ],
)
def _sc_e(p_hbm, q_hbm, srcm, dstm, out_hbm, pv, qv, eb_src, eb_dst, ob):
    cid = lax.axis_index("c")
    sid = lax.axis_index("s")
    wid = cid * 16 + sid

    pltpu.sync_copy(p_hbm, pv)
    pltpu.sync_copy(q_hbm, qv)

    def p4b(jj, _):
        base = wid * RT2 + jj * 8
        pltpu.sync_copy(srcm.at[pl.ds(base, 8), :], eb_src)
        pltpu.sync_copy(dstm.at[pl.ds(base, 8), :], eb_dst)

        def p4(j, _):
            for g in range(8):
                srcv = eb_src[j, pl.ds(g * 16, 16)]
                dstv = eb_dst[j, pl.ds(g * 16, 16)]
                ov = (plsc.load_gather(pv, [srcv])
                      + plsc.load_gather(qv, [dstv]))
                ob[j, pl.ds(g * 16, 16)] = ov
            return 0

        lax.fori_loop(0, 8, p4, 0)
        pltpu.sync_copy(ob, out_hbm.at[pl.ds(base, 8), :])
        return 0

    lax.fori_loop(0, RT2 // 8, p4b, 0)


_BLK = 1024


def _tc_s_body(x_ref, a0_ref, a1_ref, o_ref):
    o_ref[...] = x_ref[...] + a0_ref[...] + a1_ref[...]


def _tc_s(x2d, a0, a1):
    return pl.pallas_call(
        _tc_s_body,
        out_shape=jax.ShapeDtypeStruct((NP // 128, 128), jnp.float32),
    )(x2d, a0, a1)


def _tc_b_body(s_ref, w1_ref, bn1_ref, a0_ref, a1_ref, w2_ref, bn2_ref, o_ref):
    z = s_ref[...] * w1_ref[...] + bn1_ref[...] + a0_ref[0] + a1_ref[0]
    o_ref[...] = jax.lax.dot_general(
        z, w2_ref[...], (((1,), (0,)), ((), ())),
        preferred_element_type=jnp.float32,
        precision=jax.lax.Precision.HIGHEST) + bn2_ref[...]


def _tc_b(s2d, w1r, bn1r, agg2, W2, bn2r):
    grid = (NP // _BLK,)
    return pl.pallas_call(
        _tc_b_body,
        grid=grid,
        in_specs=[
            pl.BlockSpec((_BLK, 1), lambda i: (i, 0)),
            pl.BlockSpec((1, H), lambda i: (0, 0)),
            pl.BlockSpec((1, H), lambda i: (0, 0)),
            pl.BlockSpec((1, _BLK, H), lambda i: (0, i, 0)),
            pl.BlockSpec((1, _BLK, H), lambda i: (1, i, 0)),
            pl.BlockSpec((H, H), lambda i: (0, 0)),
            pl.BlockSpec((1, H), lambda i: (0, 0)),
        ],
        out_specs=pl.BlockSpec((_BLK, H), lambda i: (i, 0)),
        out_shape=jax.ShapeDtypeStruct((NP, H), jnp.float32),
    )(s2d, w1r, bn1r, agg2, agg2, W2, bn2r)


def _tc_d_body(h2_ref, a0_ref, a1_ref, g1_ref, g2_ref, dc_ref, p_ref, q_ref):
    y = h2_ref[...] + a0_ref[0] + a1_ref[0]
    p_ref[...] = jnp.sum(y * g1_ref[...], axis=1, keepdims=True)
    q_ref[...] = jnp.sum(y * g2_ref[...], axis=1, keepdims=True) + dc_ref[0, 0]


def _tc_d(h2, agg3, g1r, g2r, dc):
    grid = (NP // _BLK,)
    return pl.pallas_call(
        _tc_d_body,
        grid=grid,
        in_specs=[
            pl.BlockSpec((_BLK, H), lambda i: (i, 0)),
            pl.BlockSpec((1, _BLK, H), lambda i: (0, i, 0)),
            pl.BlockSpec((1, _BLK, H), lambda i: (1, i, 0)),
            pl.BlockSpec((1, H), lambda i: (0, 0)),
            pl.BlockSpec((1, H), lambda i: (0, 0)),
            pl.BlockSpec((1, 1), lambda i: (0, 0)),
        ],
        out_specs=[
            pl.BlockSpec((_BLK, 1), lambda i: (i, 0)),
            pl.BlockSpec((_BLK, 1), lambda i: (i, 0)),
        ],
        out_shape=[
            jax.ShapeDtypeStruct((NP, 1), jnp.float32),
            jax.ShapeDtypeStruct((NP, 1), jnp.float32),
        ],
    )(h2, agg3, agg3, g1r, g2r, dc)


def kernel(x, edge_index, edge_attr, W_em, b_em, W_le1, b_le1, W_nn1, b_nn1,
           W_le2, b_le2, W_nn2, b_nn2, W_le3, b_le3, W_nn3, b_nn3,
           W_dec, b_dec):
    src = edge_index[0].astype(jnp.int32)
    dst = edge_index[1].astype(jnp.int32)
    t = edge_attr[:, 0]

    xp = jnp.pad(x[:, 0], (0, NP - N))
    pad = EP - E
    srcm = jnp.concatenate([src, jnp.zeros((pad,), jnp.int32)]).reshape(ER, 128)
    dpad = N + (jnp.arange(pad, dtype=jnp.int32) % 32)
    dstm = jnp.concatenate([dst, dpad]).reshape(ER, 128)
    tm = jnp.concatenate([t, jnp.zeros((pad,), jnp.float32)]).reshape(ER, 128)

    a1 = (W_em @ W_le1)[0, 0]
    c1 = (b_em @ W_le1 + b_le1)[0]
    scal = jnp.stack([jnp.full((16,), a1, jnp.float32),
                      jnp.full((16,), c1, jnp.float32)])
    w1 = W_nn1[0]
    v2 = (W_em @ W_le2)[0]
    u2 = b_nn1 + b_em @ W_le2 + b_le2
    cvec2 = jnp.stack([w1, v2, u2])
    v3 = (W_em @ W_le3)[0]
    u3 = b_em @ W_le3 + b_le3
    cvec3 = jnp.stack([v3, u3])
    g1 = W_nn3 @ W_dec[:H, 0]
    g2 = W_nn3 @ W_dec[H:, 0]
    dconst = b_nn3 @ (W_dec[:H, 0] + W_dec[H:, 0]) + b_dec[0]

    agg1 = _sc_a1(xp, srcm, dstm, tm, scal)
    s2d = _tc_s(xp.reshape(NP // 128, 128),
                agg1[0, :, 0].reshape(NP // 128, 128),
                agg1[1, :, 0].reshape(NP // 128, 128))
    s_vec = s2d.reshape(NP)
    agg2 = _sc_a2(s_vec, srcm, dstm, tm, cvec2)
    h2 = _tc_b(s_vec[:, None], w1[None, :], b_nn1[None, :], agg2,
               W_nn2, b_nn2[None, :])
    agg3 = _sc_c(h2, srcm, dstm, tm, cvec3)
    pcol, qcol = _tc_d(h2, agg3, g1[None, :], g2[None, :],
                       jnp.full((1, 1), dconst, jnp.float32))
    out2d = _sc_e(pcol[:, 0], qcol[:, 0], srcm, dstm)
    return out2d.reshape(EP)[:E][:, None]
